# baseline (speedup 1.0000x reference)
import functools

import jax
import jax.numpy as jnp
from jax import lax
from jax.experimental import pallas as pl
from jax.experimental.pallas import tpu as pltpu
from jax.experimental.pallas import tpu_sc as plsc

N = 10000
E = 320000
F = 128
H = F // 2

NC = 2
NS = 16
EPT = E // NS
K = 128
CHUNKS = 160
EPT_PAD = CHUNKS * K
N_PAD = 10240
ROWS_PER_TILE = N_PAD // NS


def _sc_segment_sum(x2, src_pair, dst3, zeros):
    mesh = plsc.VectorSubcoreMesh(core_axis_name="c", subcore_axis_name="s",
                                  num_cores=NC, num_subcores=NS)

    @functools.partial(
        pl.kernel,
        out_type=jax.ShapeDtypeStruct((NC * N_PAD, H), jnp.float32),
        mesh=mesh,
        scratch_types=[
            pltpu.VMEM((CHUNKS, K), jnp.int32),
            pltpu.VMEM((CHUNKS, K), jnp.int32),
            [pltpu.VMEM((K, H), jnp.float32)] * 4,
            pltpu.VMEM_SHARED((N_PAD, H), jnp.float32),
            [pltpu.SemaphoreType.DMA] * 4,
            [pltpu.SemaphoreType.DMA] * 4,
        ],
        compiler_params=pltpu.CompilerParams(use_tc_tiling_on_sc=False),
    )
    def seg_kernel(x_hbm, src_hbm, dst_hbm, zeros_hbm, out_hbm,
                   sidx, didx, rows, acc, gsem, ssem):
        cid = lax.axis_index("c")
        sid = lax.axis_index("s")

        pltpu.sync_copy(src_hbm.at[cid, sid], sidx)
        pltpu.sync_copy(dst_hbm.at[sid], didx)
        pltpu.sync_copy(zeros_hbm, acc.at[pl.ds(sid * ROWS_PER_TILE,
                                                ROWS_PER_TILE)])
        plsc.subcore_barrier()

        def wait_gather(r):
            pltpu.make_async_copy(x_hbm.at[pl.ds(0, K)], rows[r],
                                  gsem[r]).wait()

        def wait_scatter(r):
            pltpu.make_async_copy(rows[r], acc.at[didx.at[0]],
                                  ssem[r]).wait()

        def chunk_body(i, r, do_ss, do_gather):
            e = (r + 2) % 4
            wait_gather(r)
            pltpu.async_copy(rows[r], acc.at[didx.at[i]], ssem[r], add=True)
            if do_ss:
                wait_scatter(e)
            if do_gather:
                pltpu.async_copy(x_hbm.at[sidx.at[i + 2]], rows[e], gsem[e])

        pltpu.async_copy(x_hbm.at[sidx.at[0]], rows[0], gsem[0])
        pltpu.async_copy(x_hbm.at[sidx.at[1]], rows[1], gsem[1])

        def step(c, r, do_gather):
            wait_gather(r)
            pltpu.sync_copy(rows[r], acc.at[didx.at[c]], add=True)
            if do_gather:
                pltpu.async_copy(x_hbm.at[sidx.at[c + 2]], rows[r], gsem[r])

        def body(j, _):
            step(2 * j, 0, True)
            step(2 * j + 1, 1, True)
            return 0

        lax.fori_loop(0, CHUNKS // 2 - 1, body, 0)
        step(CHUNKS - 2, 0, False)
        step(CHUNKS - 1, 1, False)
        plsc.subcore_barrier()

        row0 = sid * ROWS_PER_TILE
        pltpu.sync_copy(acc.at[pl.ds(row0, ROWS_PER_TILE)],
                        out_hbm.at[pl.ds(cid * N_PAD + row0, ROWS_PER_TILE)])

    return seg_kernel(x2, src_pair, dst3, zeros)


def _tc_linear(x, parts, in_a, out_a, Wx, Ws0, Ws1, b):
    B = 1000
    grid = N // B

    def body(x_ref, p_ref, ina_ref, outa_ref, wx_ref, ws0_ref, ws1_ref,
             b_ref, o_ref):
        mm = functools.partial(jnp.dot, preferred_element_type=jnp.float32,
                               precision=lax.Precision.HIGHEST)
        res = (mm(x_ref[...], wx_ref[...])
               + mm(p_ref[0], ws0_ref[...])
               + mm(p_ref[1], ws1_ref[...])
               + b_ref[...])
        o_ref[...] = ina_ref[...] * res[:, :F] + outa_ref[...] * res[:, F:]

    return pl.pallas_call(
        body,
        grid=(grid,),
        in_specs=[
            pl.BlockSpec((B, F), lambda i: (i, 0)),
            pl.BlockSpec((NC, B, H), lambda i: (0, i, 0)),
            pl.BlockSpec((B, 1), lambda i: (i, 0)),
            pl.BlockSpec((B, 1), lambda i: (i, 0)),
            pl.BlockSpec((F, 2 * F), lambda i: (0, 0)),
            pl.BlockSpec((H, 2 * F), lambda i: (0, 0)),
            pl.BlockSpec((H, 2 * F), lambda i: (0, 0)),
            pl.BlockSpec((1, 2 * F), lambda i: (0, 0)),
        ],
        out_specs=pl.BlockSpec((B, F), lambda i: (i, 0)),
        out_shape=jax.ShapeDtypeStruct((N, F), jnp.float32),
    )(x, parts, in_a, out_a, Wx, Ws0, Ws1, b)


def kernel(x, edge_index, in_a, out_a, W_in, b_in, W_out, b_out):
    pad = EPT_PAD - EPT
    src = jnp.pad(edge_index[0].reshape(NS, EPT), ((0, 0), (0, pad)))
    dst = jnp.pad(edge_index[1].reshape(NS, EPT), ((0, 0), (0, pad)),
                  constant_values=N)
    x2 = x.reshape(N, NC, H).swapaxes(0, 1).reshape(NC * N, H)
    src_pair = jnp.stack([src, src + N]).reshape(NC, NS, CHUNKS, K)
    dst3 = dst.reshape(NS, CHUNKS, K)
    zeros = jnp.zeros((ROWS_PER_TILE, H), jnp.float32)
    parts = _sc_segment_sum(x2, src_pair, dst3, zeros).reshape(NC, N_PAD, H)

    Wx = jnp.concatenate([W_in[:, :F].T, W_out[:, :F].T], axis=1)
    Ws = jnp.concatenate([W_in[:, F:].T, W_out[:, F:].T], axis=1)
    b = jnp.concatenate([b_in, b_out]).reshape(1, 2 * F)
    return _tc_linear(x, parts, in_a, out_a, Wx, Ws[:H], Ws[H:], b)


# speedup vs baseline: 1.2850x; 1.2850x over previous
import functools

import jax
import jax.numpy as jnp
from jax import lax
from jax.experimental import pallas as pl
from jax.experimental.pallas import tpu as pltpu
from jax.experimental.pallas import tpu_sc as plsc

N = 10000
E = 320000
F = 128
H = F // 2

NC = 2
NS = 16
EPT = E // NS
K = 80
CHUNKS = 252
EPT_PAD = CHUNKS * K
N_PAD = 10240
ROWS_PER_TILE = N_PAD // NS


def _sc_segment_sum(x2, src_pair, dst3, zeros):
    mesh = plsc.VectorSubcoreMesh(core_axis_name="c", subcore_axis_name="s",
                                  num_cores=NC, num_subcores=NS)

    @functools.partial(
        pl.kernel,
        out_type=jax.ShapeDtypeStruct((NC * N_PAD, H), jnp.float32),
        mesh=mesh,
        scratch_types=[
            pltpu.VMEM((CHUNKS, K), jnp.int32),
            pltpu.VMEM((CHUNKS, K), jnp.int32),
            [pltpu.VMEM((K, H), jnp.float32)] * 4,
            pltpu.VMEM_SHARED((N_PAD, H), jnp.float32),
            [pltpu.SemaphoreType.DMA] * 4,
            [pltpu.SemaphoreType.DMA] * 4,
        ],
        compiler_params=pltpu.CompilerParams(use_tc_tiling_on_sc=False),
    )
    def seg_kernel(x_hbm, src_hbm, dst_hbm, zeros_hbm, out_hbm,
                   sidx, didx, rows, acc, gsem, ssem):
        cid = lax.axis_index("c")
        sid = lax.axis_index("s")

        pltpu.sync_copy(src_hbm.at[cid, sid], sidx)
        pltpu.sync_copy(dst_hbm.at[sid], didx)
        pltpu.sync_copy(zeros_hbm, acc.at[pl.ds(sid * ROWS_PER_TILE,
                                                ROWS_PER_TILE)])
        plsc.subcore_barrier()

        def wait_gather(r):
            pltpu.make_async_copy(x_hbm.at[pl.ds(0, K)], rows[r],
                                  gsem[r]).wait()

        def wait_scatter(r):
            pltpu.make_async_copy(rows[r], acc.at[didx.at[0]],
                                  ssem[r]).wait()

        def chunk_body(i, r, do_ss, do_gather):
            e = (r + 2) % 4
            wait_gather(r)
            pltpu.async_copy(rows[r], acc.at[didx.at[i]], ssem[r], add=True)
            if do_ss:
                wait_scatter(e)
            if do_gather:
                pltpu.async_copy(x_hbm.at[sidx.at[i + 2]], rows[e], gsem[e])

        pltpu.async_copy(x_hbm.at[sidx.at[0]], rows[0], gsem[0])
        pltpu.async_copy(x_hbm.at[sidx.at[1]], rows[1], gsem[1])
        chunk_body(0, 0, False, True)
        chunk_body(1, 1, False, True)
        chunk_body(2, 2, True, True)
        chunk_body(3, 3, True, True)

        def body(m, _):
            for r in range(4):
                chunk_body(4 * m + r, r, True, True)
            return 0

        lax.fori_loop(1, CHUNKS // 4 - 1, body, 0)

        chunk_body(CHUNKS - 4, 0, True, True)
        chunk_body(CHUNKS - 3, 1, True, True)
        chunk_body(CHUNKS - 2, 2, True, False)
        chunk_body(CHUNKS - 1, 3, True, False)
        wait_scatter(2)
        wait_scatter(3)
        plsc.subcore_barrier()

        row0 = sid * ROWS_PER_TILE
        pltpu.sync_copy(acc.at[pl.ds(row0, ROWS_PER_TILE)],
                        out_hbm.at[pl.ds(cid * N_PAD + row0, ROWS_PER_TILE)])

    return seg_kernel(x2, src_pair, dst3, zeros)


def _tc_linear(x, parts, in_a, out_a, Wx, Ws0, Ws1, b):
    B = 1000
    grid = N // B

    def body(x_ref, p_ref, ina_ref, outa_ref, wx_ref, ws0_ref, ws1_ref,
             b_ref, o_ref):
        mm = functools.partial(jnp.dot, preferred_element_type=jnp.float32,
                               precision=lax.Precision.HIGHEST)
        res = (mm(x_ref[...], wx_ref[...])
               + mm(p_ref[0], ws0_ref[...])
               + mm(p_ref[1], ws1_ref[...])
               + b_ref[...])
        o_ref[...] = ina_ref[...] * res[:, :F] + outa_ref[...] * res[:, F:]

    return pl.pallas_call(
        body,
        grid=(grid,),
        in_specs=[
            pl.BlockSpec((B, F), lambda i: (i, 0)),
            pl.BlockSpec((NC, B, H), lambda i: (0, i, 0)),
            pl.BlockSpec((B, 1), lambda i: (i, 0)),
            pl.BlockSpec((B, 1), lambda i: (i, 0)),
            pl.BlockSpec((F, 2 * F), lambda i: (0, 0)),
            pl.BlockSpec((H, 2 * F), lambda i: (0, 0)),
            pl.BlockSpec((H, 2 * F), lambda i: (0, 0)),
            pl.BlockSpec((1, 2 * F), lambda i: (0, 0)),
        ],
        out_specs=pl.BlockSpec((B, F), lambda i: (i, 0)),
        out_shape=jax.ShapeDtypeStruct((N, F), jnp.float32),
    )(x, parts, in_a, out_a, Wx, Ws0, Ws1, b)


def kernel(x, edge_index, in_a, out_a, W_in, b_in, W_out, b_out):
    pad = EPT_PAD - EPT
    src = jnp.pad(edge_index[0].reshape(NS, EPT), ((0, 0), (0, pad)))
    dst = jnp.pad(edge_index[1].reshape(NS, EPT), ((0, 0), (0, pad)),
                  constant_values=N)
    x2 = x.reshape(N, NC, H).swapaxes(0, 1).reshape(NC * N, H)
    src_pair = jnp.stack([src, src + N]).reshape(NC, NS, CHUNKS, K)
    dst3 = dst.reshape(NS, CHUNKS, K)
    zeros = jnp.zeros((ROWS_PER_TILE, H), jnp.float32)
    parts = _sc_segment_sum(x2, src_pair, dst3, zeros).reshape(NC, N_PAD, H)

    Wx = jnp.concatenate([W_in[:, :F].T, W_out[:, :F].T], axis=1)
    Ws = jnp.concatenate([W_in[:, F:].T, W_out[:, F:].T], axis=1)
    b = jnp.concatenate([b_in, b_out]).reshape(1, 2 * F)
    return _tc_linear(x, parts, in_a, out_a, Wx, Ws[:H], Ws[H:], b)


# speedup vs baseline: 1.4419x; 1.1221x over previous
import functools

import jax
import jax.numpy as jnp
from jax import lax
from jax.experimental import pallas as pl
from jax.experimental.pallas import tpu as pltpu
from jax.experimental.pallas import tpu_sc as plsc

N = 10000
E = 320000
F = 128
H = F // 2

NC = 2
NS = 16
EPT = E // NS
K = 80
CHUNKS = 250
EPT_PAD = CHUNKS * K
N_PAD = 10240
ROWS_PER_TILE = N_PAD // NS


def _sc_segment_sum(x2, src_pair, dst3, zeros):
    mesh = plsc.VectorSubcoreMesh(core_axis_name="c", subcore_axis_name="s",
                                  num_cores=NC, num_subcores=NS)

    @functools.partial(
        pl.kernel,
        out_type=jax.ShapeDtypeStruct((NC * N_PAD, H), jnp.float32),
        mesh=mesh,
        scratch_types=[
            pltpu.VMEM((CHUNKS, K), jnp.int32),
            pltpu.VMEM((CHUNKS, K), jnp.int32),
            [pltpu.VMEM((K, H), jnp.float32)] * 4,
            pltpu.VMEM_SHARED((N_PAD, H), jnp.float32),
            [pltpu.SemaphoreType.DMA] * 4,
            [pltpu.SemaphoreType.DMA] * 4,
        ],
        compiler_params=pltpu.CompilerParams(use_tc_tiling_on_sc=False),
    )
    def seg_kernel(x_hbm, src_hbm, dst_hbm, zeros_hbm, out_hbm,
                   sidx, didx, rows, acc, gsem, ssem):
        cid = lax.axis_index("c")
        sid = lax.axis_index("s")

        pltpu.sync_copy(src_hbm.at[cid, sid], sidx)
        pltpu.sync_copy(dst_hbm.at[sid], didx)
        pltpu.sync_copy(zeros_hbm, acc.at[pl.ds(sid * ROWS_PER_TILE,
                                                ROWS_PER_TILE)])
        plsc.subcore_barrier()

        def wait_gather(r):
            pltpu.make_async_copy(x_hbm.at[pl.ds(0, K)], rows[r],
                                  gsem[r]).wait()

        def wait_scatter(r):
            pltpu.make_async_copy(rows[r], acc.at[didx.at[0]],
                                  ssem[r]).wait()

        def chunk_body(i, r, do_ss, do_gather):
            e = (r + 2) % 4
            wait_gather(r)
            pltpu.async_copy(rows[r], acc.at[didx.at[i]], ssem[r], add=True)
            if do_ss:
                wait_scatter(e)
            if do_gather:
                pltpu.async_copy(x_hbm.at[sidx.at[i + 2]], rows[e], gsem[e])

        pltpu.async_copy(x_hbm.at[sidx.at[0]], rows[0], gsem[0])
        pltpu.async_copy(x_hbm.at[sidx.at[1]], rows[1], gsem[1])

        def step(c, r, do_gather):
            wait_gather(r)
            pltpu.sync_copy(rows[r], acc.at[didx.at[c]], add=True)
            if do_gather:
                pltpu.async_copy(x_hbm.at[sidx.at[c + 2]], rows[r], gsem[r])

        def body(j, _):
            step(2 * j, 0, True)
            step(2 * j + 1, 1, True)
            return 0

        lax.fori_loop(0, CHUNKS // 2 - 1, body, 0)
        step(CHUNKS - 2, 0, False)
        step(CHUNKS - 1, 1, False)
        plsc.subcore_barrier()

        row0 = sid * ROWS_PER_TILE
        pltpu.sync_copy(acc.at[pl.ds(row0, ROWS_PER_TILE)],
                        out_hbm.at[pl.ds(cid * N_PAD + row0, ROWS_PER_TILE)])

    return seg_kernel(x2, src_pair, dst3, zeros)


def _tc_linear(x, parts, in_a, out_a, Wx, Ws0, Ws1, b):
    B = 1000
    grid = N // B

    def body(x_ref, p_ref, ina_ref, outa_ref, wx_ref, ws0_ref, ws1_ref,
             b_ref, o_ref):
        mm = functools.partial(jnp.dot, preferred_element_type=jnp.float32,
                               precision=lax.Precision.HIGHEST)
        res = (mm(x_ref[...], wx_ref[...])
               + mm(p_ref[0], ws0_ref[...])
               + mm(p_ref[1], ws1_ref[...])
               + b_ref[...])
        o_ref[...] = ina_ref[...] * res[:, :F] + outa_ref[...] * res[:, F:]

    return pl.pallas_call(
        body,
        grid=(grid,),
        in_specs=[
            pl.BlockSpec((B, F), lambda i: (i, 0)),
            pl.BlockSpec((NC, B, H), lambda i: (0, i, 0)),
            pl.BlockSpec((B, 1), lambda i: (i, 0)),
            pl.BlockSpec((B, 1), lambda i: (i, 0)),
            pl.BlockSpec((F, 2 * F), lambda i: (0, 0)),
            pl.BlockSpec((H, 2 * F), lambda i: (0, 0)),
            pl.BlockSpec((H, 2 * F), lambda i: (0, 0)),
            pl.BlockSpec((1, 2 * F), lambda i: (0, 0)),
        ],
        out_specs=pl.BlockSpec((B, F), lambda i: (i, 0)),
        out_shape=jax.ShapeDtypeStruct((N, F), jnp.float32),
    )(x, parts, in_a, out_a, Wx, Ws0, Ws1, b)


def kernel(x, edge_index, in_a, out_a, W_in, b_in, W_out, b_out):
    pad = EPT_PAD - EPT
    src = jnp.pad(edge_index[0].reshape(NS, EPT), ((0, 0), (0, pad)))
    dst = jnp.pad(edge_index[1].reshape(NS, EPT), ((0, 0), (0, pad)),
                  constant_values=N)
    x2 = x.reshape(N, NC, H).swapaxes(0, 1).reshape(NC * N, H)
    src_pair = jnp.stack([src, src + N]).reshape(NC, NS, CHUNKS, K)
    dst3 = dst.reshape(NS, CHUNKS, K)
    zeros = jnp.zeros((ROWS_PER_TILE, H), jnp.float32)
    parts = _sc_segment_sum(x2, src_pair, dst3, zeros).reshape(NC, N_PAD, H)

    Wx = jnp.concatenate([W_in[:, :F].T, W_out[:, :F].T], axis=1)
    Ws = jnp.concatenate([W_in[:, F:].T, W_out[:, F:].T], axis=1)
    b = jnp.concatenate([b_in, b_out]).reshape(1, 2 * F)
    return _tc_linear(x, parts, in_a, out_a, Wx, Ws[:H], Ws[H:], b)
